# baseline (speedup 1.0000x reference)
import functools

import jax
import jax.numpy as jnp
from jax.experimental import pallas as pl
from jax.experimental.pallas import tpu as pltpu

D_MODEL = 768
POS_DIM = 2
K_CAND = 24
K_TOP = 8
KEEP = 0.9
SEQ = 2048
T_BLK = 64


def _top8_threshold(eg):
    v = eg
    cum = jnp.zeros(eg.shape[:1] + (1,), jnp.float32)
    thr = jnp.zeros(eg.shape[:1] + (1,), jnp.float32)
    m1 = None
    for j in range(K_TOP):
        m = jnp.max(v, axis=1, keepdims=True)
        if j == 0:
            m1 = m
        eqm = v == m
        c = jnp.sum(jnp.where(eqm, 1.0, 0.0), axis=1, keepdims=True)
        thr = jnp.where((cum < K_TOP) & (cum + c >= K_TOP), m, thr)
        cum = cum + c
        if j < K_TOP - 1:
            v = jnp.where(eqm, -jnp.inf, v)
    return thr, m1


def _threshold_gate(scores, tau):
    raw = scores - tau
    gate = jnp.where(raw > 0, raw, 1e-08 * jnp.exp(raw))
    eg = jnp.exp(gate) - 1.0
    thr, m1 = _top8_threshold(eg)
    eg = jnp.where(eg >= thr, eg, 0.0)
    gsum = jnp.sum(eg, axis=1, keepdims=True) + 1e-08
    gstr = jnp.tanh(m1)
    return eg / gsum * gstr


def _extract(pos, npt, scores, N, unroll):
    d0 = pos[:, 0:1] - npt[0:1, :]
    d1 = pos[:, 1:2] - npt[1:2, :]
    dist = d0 * d0 + d1 * d1
    iota_nf = jax.lax.broadcasted_iota(
        jnp.int32, (T_BLK, N), 1).astype(jnp.float32)
    iota_k = jax.lax.broadcasted_iota(jnp.int32, (T_BLK, K_CAND), 1)
    bigf = jnp.float32(N)

    def step(j, carry):
        d, acc_i, acc_d, acc_s = carry
        for u in range(unroll):
            m = jnp.min(d, axis=1, keepdims=True)
            keyf = jnp.where(d == m, iota_nf, bigf)
            idxf = jnp.min(keyf, axis=1, keepdims=True)
            sel = keyf == idxf
            s = jnp.sum(jnp.where(sel, scores, 0.0), axis=1, keepdims=True)
            d = jnp.where(sel, jnp.inf, d)
            lane = iota_k == j * unroll + u
            acc_i = jnp.where(lane, idxf, acc_i)
            acc_d = jnp.where(lane, m, acc_d)
            acc_s = jnp.where(lane, s, acc_s)
        return d, acc_i, acc_d, acc_s

    init = (dist,
            jnp.zeros((T_BLK, K_CAND), jnp.float32),
            jnp.zeros((T_BLK, K_CAND), jnp.float32),
            jnp.zeros((T_BLK, K_CAND), jnp.float32))
    _, cand_if, cand_d, cand_s = jax.lax.fori_loop(
        0, K_CAND // unroll, step, init)
    return cand_if, cand_d, cand_s


def _body(x_ref, pqk_ref, pv_ref, pkn_ref, nqk_ref, nv_ref, nkn_ref,
          npqk_ref, npv_ref, npkn_ref, wta_ref, bta_ref, wtk_ref, btk_ref,
          iqk_ref, iv_ref, ikn_ref, la_ref, lk_ref,
          gq_ref, gk_ref, gv_ref, gkn_ref):
    i = pl.program_id(0)
    x = x_ref[...]
    xs = x / KEEP

    def mm(a, b_ref):
        return jax.lax.dot_general(
            a, b_ref[...], (((1,), (1,)), ((), ())),
            preferred_element_type=jnp.float32)

    tau_a = jax.lax.dot_general(
        x, wta_ref[...], (((1,), (0,)), ((), ())),
        preferred_element_type=jnp.float32) + bta_ref[...]
    tau_k = jax.lax.dot_general(
        x, wtk_ref[...], (((1,), (0,)), ((), ())),
        preferred_element_type=jnp.float32) + btk_ref[...]

    s_qk = mm(xs, nqk_ref)
    ci, cd, cs = _extract(pqk_ref[...], npqk_ref[...], s_qk, 2048, 12)
    iqk_ref[...] = ci.astype(jnp.int32)
    g_q = _threshold_gate(cs, tau_a[:, 0:1])
    gq_ref[...] = g_q
    gk_ref[...] = _threshold_gate(cs, tau_a[:, 1:2])
    part_a = jnp.sum(g_q * cd, axis=(0, 1), keepdims=True)

    s_v = mm(xs, nv_ref)
    ci, cd, cs = _extract(pv_ref[...], npv_ref[...], s_v, 2048, 12)
    iv_ref[...] = ci.astype(jnp.int32)
    g_v = _threshold_gate(cs, tau_a[:, 2:3])
    gv_ref[...] = g_v
    part_a += jnp.sum(g_v * cd, axis=(0, 1), keepdims=True)

    s_kn = mm(xs, nkn_ref)
    ci, cd, cs = _extract(pkn_ref[...], npkn_ref[...], s_kn, 4096, 12)
    ikn_ref[...] = ci.astype(jnp.int32)
    g_kn = _threshold_gate(cs, tau_k)
    gkn_ref[...] = g_kn
    part_k = jnp.sum(g_kn * cd, axis=(0, 1), keepdims=True)

    @pl.when(i == 0)
    def _():
        la_ref[...] = jnp.zeros((1, 1), jnp.float32)
        lk_ref[...] = jnp.zeros((1, 1), jnp.float32)

    la_ref[...] += part_a
    lk_ref[...] += part_k


def kernel(x, qk_neurons, v_neurons, know_neurons, neuron_pos, W_pos_qk,
           b_pos_qk, W_pos_v, b_pos_v, W_pos_know, b_pos_know, W_tau_attn,
           b_tau_attn, W_tau_know, b_tau_know, deterministic):
    del deterministic
    n_qk = qk_neurons.shape[0]
    n_v = v_neurons.shape[0]
    x2d = x.reshape(SEQ, D_MODEL)

    pos_qk = (x @ W_pos_qk + b_pos_qk).reshape(SEQ, POS_DIM)
    pos_v = (x @ W_pos_v + b_pos_v).reshape(SEQ, POS_DIM)
    pos_know = (x @ W_pos_know + b_pos_know).reshape(SEQ, POS_DIM)

    npt_qk = neuron_pos[:n_qk].T
    npt_v = neuron_pos[n_qk:n_qk + n_v].T
    npt_kn = neuron_pos[n_qk + n_v:].T
    bta = b_tau_attn.reshape(1, 3)
    btk = b_tau_know.reshape(1, 1)

    grid = (SEQ // T_BLK,)
    tok = lambda i: (i, 0)
    rep = lambda i: (0, 0)
    sd = jax.ShapeDtypeStruct
    out_shape = [
        sd((SEQ, K_CAND), jnp.int32), sd((SEQ, K_CAND), jnp.int32),
        sd((SEQ, K_CAND), jnp.int32),
        sd((1, 1), jnp.float32), sd((1, 1), jnp.float32),
        sd((SEQ, K_CAND), jnp.float32), sd((SEQ, K_CAND), jnp.float32),
        sd((SEQ, K_CAND), jnp.float32), sd((SEQ, K_CAND), jnp.float32),
    ]
    in_specs = [
        pl.BlockSpec((T_BLK, D_MODEL), tok),
        pl.BlockSpec((T_BLK, POS_DIM), tok),
        pl.BlockSpec((T_BLK, POS_DIM), tok),
        pl.BlockSpec((T_BLK, POS_DIM), tok),
        pl.BlockSpec((2048, D_MODEL), rep),
        pl.BlockSpec((2048, D_MODEL), rep),
        pl.BlockSpec((4096, D_MODEL), rep),
        pl.BlockSpec((POS_DIM, 2048), rep),
        pl.BlockSpec((POS_DIM, 2048), rep),
        pl.BlockSpec((POS_DIM, 4096), rep),
        pl.BlockSpec((D_MODEL, 3), rep),
        pl.BlockSpec((1, 3), rep),
        pl.BlockSpec((D_MODEL, 1), rep),
        pl.BlockSpec((1, 1), rep),
    ]
    out_specs = [
        pl.BlockSpec((T_BLK, K_CAND), tok), pl.BlockSpec((T_BLK, K_CAND), tok),
        pl.BlockSpec((T_BLK, K_CAND), tok),
        pl.BlockSpec((1, 1), rep), pl.BlockSpec((1, 1), rep),
        pl.BlockSpec((T_BLK, K_CAND), tok), pl.BlockSpec((T_BLK, K_CAND), tok),
        pl.BlockSpec((T_BLK, K_CAND), tok), pl.BlockSpec((T_BLK, K_CAND), tok),
    ]
    fn = pl.pallas_call(
        _body,
        grid=grid,
        in_specs=in_specs,
        out_specs=out_specs,
        out_shape=out_shape,
        compiler_params=pltpu.CompilerParams(
            dimension_semantics=("arbitrary",)),
    )
    (idx_qk, idx_v, idx_know, loss_a, loss_k,
     gate_q, gate_k, gate_v, gate_know) = fn(
        x2d, pos_qk, pos_v, pos_know, qk_neurons, v_neurons, know_neurons,
        npt_qk, npt_v, npt_kn, W_tau_attn, bta, W_tau_know, btk)

    denom = jnp.float32(SEQ * K_CAND)
    pos_loss_attn = loss_a[0, 0] / denom
    pos_loss_know = loss_k[0, 0] / denom

    to3 = lambda a: a.reshape(1, SEQ, K_CAND)
    return (to3(gate_q), to3(gate_k), to3(gate_v), to3(idx_qk), to3(idx_v),
            pos_loss_attn, to3(gate_know), to3(idx_know), pos_loss_know)


# speedup vs baseline: 1.1743x; 1.1743x over previous
import functools

import jax
import jax.numpy as jnp
from jax.experimental import pallas as pl
from jax.experimental.pallas import tpu as pltpu

D_MODEL = 768
POS_DIM = 2
K_CAND = 24
K_TOP = 8
KEEP = 0.9
SEQ = 2048
T_BLK = 128


def _top8_threshold(eg):
    v = eg
    cum = jnp.zeros(eg.shape[:1] + (1,), jnp.float32)
    thr = jnp.zeros(eg.shape[:1] + (1,), jnp.float32)
    m1 = None
    for j in range(K_TOP):
        m = jnp.max(v, axis=1, keepdims=True)
        if j == 0:
            m1 = m
        eqm = v == m
        c = jnp.sum(jnp.where(eqm, 1.0, 0.0), axis=1, keepdims=True)
        thr = jnp.where((cum < K_TOP) & (cum + c >= K_TOP), m, thr)
        cum = cum + c
        if j < K_TOP - 1:
            v = jnp.where(eqm, -jnp.inf, v)
    return thr, m1


def _threshold_gate(scores, tau):
    raw = scores - tau
    gate = jnp.where(raw > 0, raw, 1e-08 * jnp.exp(raw))
    eg = jnp.exp(gate) - 1.0
    thr, m1 = _top8_threshold(eg)
    eg = jnp.where(eg >= thr, eg, 0.0)
    gsum = jnp.sum(eg, axis=1, keepdims=True) + 1e-08
    gstr = jnp.tanh(m1)
    return eg / gsum * gstr


def _extract(pos, npt, scores, N, unroll, side_fn=None):
    d0 = pos[:, 0:1] - npt[0:1, :]
    d1 = pos[:, 1:2] - npt[1:2, :]
    dist = d0 * d0 + d1 * d1
    iota_nf = jax.lax.broadcasted_iota(
        jnp.int32, (T_BLK, N), 1).astype(jnp.float32)
    iota_k = jax.lax.broadcasted_iota(jnp.int32, (T_BLK, K_CAND), 1)
    bigf = jnp.float32(N)

    def step(j, carry):
        d, acc_i, acc_d, acc_s = carry
        if side_fn is not None:
            side_fn(j)
        for u in range(unroll):
            m = jnp.min(d, axis=1, keepdims=True)
            keyf = jnp.where(d == m, iota_nf, bigf)
            idxf = jnp.min(keyf, axis=1, keepdims=True)
            sel = keyf == idxf
            s = jnp.sum(jnp.where(sel, scores, 0.0), axis=1, keepdims=True)
            d = jnp.where(sel, jnp.inf, d)
            lane = iota_k == j * unroll + u
            acc_i = jnp.where(lane, idxf, acc_i)
            acc_d = jnp.where(lane, m, acc_d)
            acc_s = jnp.where(lane, s, acc_s)
        return d, acc_i, acc_d, acc_s

    init = (dist,
            jnp.zeros((T_BLK, K_CAND), jnp.float32),
            jnp.zeros((T_BLK, K_CAND), jnp.float32),
            jnp.zeros((T_BLK, K_CAND), jnp.float32))
    _, cand_if, cand_d, cand_s = jax.lax.fori_loop(
        0, K_CAND // unroll, step, init)
    return cand_if, cand_d, cand_s


def _body(x_ref, pqk_ref, pv_ref, pkn_ref, nqk_ref, nv_ref, nkn_ref,
          npqk_ref, npv_ref, npkn_ref, wta_ref, bta_ref, wtk_ref, btk_ref,
          iqk_ref, iv_ref, ikn_ref, la_ref, lk_ref,
          gq_ref, gk_ref, gv_ref, gkn_ref, sv_scr, skn_scr):
    i = pl.program_id(0)
    x = x_ref[...]
    xs = x / KEEP

    def mm(a, b_ref):
        return jax.lax.dot_general(
            a, b_ref[...], (((1,), (1,)), ((), ())),
            preferred_element_type=jnp.float32)

    tau_a = jax.lax.dot_general(
        x, wta_ref[...], (((1,), (0,)), ((), ())),
        preferred_element_type=jnp.float32) + bta_ref[...]
    tau_k = jax.lax.dot_general(
        x, wtk_ref[...], (((1,), (0,)), ((), ())),
        preferred_element_type=jnp.float32) + btk_ref[...]

    def side_v(j):
        sv_scr[:, pl.ds(j * 1024, 1024)] = jax.lax.dot_general(
            xs, nv_ref[pl.ds(j * 1024, 1024), :], (((1,), (1,)), ((), ())),
            preferred_element_type=jnp.float32)

    def side_kn(j):
        skn_scr[:, pl.ds(j * 2048, 2048)] = jax.lax.dot_general(
            xs, nkn_ref[pl.ds(j * 2048, 2048), :], (((1,), (1,)), ((), ())),
            preferred_element_type=jnp.float32)

    s_qk = mm(xs, nqk_ref)
    ci, cd, cs = _extract(pqk_ref[...], npqk_ref[...], s_qk, 2048, 12,
                          side_fn=side_v)
    iqk_ref[...] = ci.astype(jnp.int32)
    g_q = _threshold_gate(cs, tau_a[:, 0:1])
    gq_ref[...] = g_q
    gk_ref[...] = _threshold_gate(cs, tau_a[:, 1:2])
    part_a = jnp.sum(g_q * cd, axis=(0, 1), keepdims=True)

    s_v = sv_scr[...]
    ci, cd, cs = _extract(pv_ref[...], npv_ref[...], s_v, 2048, 12,
                          side_fn=side_kn)
    iv_ref[...] = ci.astype(jnp.int32)
    g_v = _threshold_gate(cs, tau_a[:, 2:3])
    gv_ref[...] = g_v
    part_a += jnp.sum(g_v * cd, axis=(0, 1), keepdims=True)

    s_kn = skn_scr[...]
    ci, cd, cs = _extract(pkn_ref[...], npkn_ref[...], s_kn, 4096, 12)
    ikn_ref[...] = ci.astype(jnp.int32)
    g_kn = _threshold_gate(cs, tau_k)
    gkn_ref[...] = g_kn
    part_k = jnp.sum(g_kn * cd, axis=(0, 1), keepdims=True)

    @pl.when(i == 0)
    def _():
        la_ref[...] = jnp.zeros((1, 1), jnp.float32)
        lk_ref[...] = jnp.zeros((1, 1), jnp.float32)

    la_ref[...] += part_a
    lk_ref[...] += part_k


def kernel(x, qk_neurons, v_neurons, know_neurons, neuron_pos, W_pos_qk,
           b_pos_qk, W_pos_v, b_pos_v, W_pos_know, b_pos_know, W_tau_attn,
           b_tau_attn, W_tau_know, b_tau_know, deterministic):
    del deterministic
    n_qk = qk_neurons.shape[0]
    n_v = v_neurons.shape[0]
    x2d = x.reshape(SEQ, D_MODEL)

    pos_qk = (x @ W_pos_qk + b_pos_qk).reshape(SEQ, POS_DIM)
    pos_v = (x @ W_pos_v + b_pos_v).reshape(SEQ, POS_DIM)
    pos_know = (x @ W_pos_know + b_pos_know).reshape(SEQ, POS_DIM)

    npt_qk = neuron_pos[:n_qk].T
    npt_v = neuron_pos[n_qk:n_qk + n_v].T
    npt_kn = neuron_pos[n_qk + n_v:].T
    bta = b_tau_attn.reshape(1, 3)
    btk = b_tau_know.reshape(1, 1)

    grid = (SEQ // T_BLK,)
    tok = lambda i: (i, 0)
    rep = lambda i: (0, 0)
    sd = jax.ShapeDtypeStruct
    out_shape = [
        sd((SEQ, K_CAND), jnp.int32), sd((SEQ, K_CAND), jnp.int32),
        sd((SEQ, K_CAND), jnp.int32),
        sd((1, 1), jnp.float32), sd((1, 1), jnp.float32),
        sd((SEQ, K_CAND), jnp.float32), sd((SEQ, K_CAND), jnp.float32),
        sd((SEQ, K_CAND), jnp.float32), sd((SEQ, K_CAND), jnp.float32),
    ]
    in_specs = [
        pl.BlockSpec((T_BLK, D_MODEL), tok),
        pl.BlockSpec((T_BLK, POS_DIM), tok),
        pl.BlockSpec((T_BLK, POS_DIM), tok),
        pl.BlockSpec((T_BLK, POS_DIM), tok),
        pl.BlockSpec((2048, D_MODEL), rep),
        pl.BlockSpec((2048, D_MODEL), rep),
        pl.BlockSpec((4096, D_MODEL), rep),
        pl.BlockSpec((POS_DIM, 2048), rep),
        pl.BlockSpec((POS_DIM, 2048), rep),
        pl.BlockSpec((POS_DIM, 4096), rep),
        pl.BlockSpec((D_MODEL, 3), rep),
        pl.BlockSpec((1, 3), rep),
        pl.BlockSpec((D_MODEL, 1), rep),
        pl.BlockSpec((1, 1), rep),
    ]
    out_specs = [
        pl.BlockSpec((T_BLK, K_CAND), tok), pl.BlockSpec((T_BLK, K_CAND), tok),
        pl.BlockSpec((T_BLK, K_CAND), tok),
        pl.BlockSpec((1, 1), rep), pl.BlockSpec((1, 1), rep),
        pl.BlockSpec((T_BLK, K_CAND), tok), pl.BlockSpec((T_BLK, K_CAND), tok),
        pl.BlockSpec((T_BLK, K_CAND), tok), pl.BlockSpec((T_BLK, K_CAND), tok),
    ]
    fn = pl.pallas_call(
        _body,
        grid=grid,
        in_specs=in_specs,
        out_specs=out_specs,
        out_shape=out_shape,
        scratch_shapes=[pltpu.VMEM((T_BLK, 2048), jnp.float32),
                        pltpu.VMEM((T_BLK, 4096), jnp.float32)],
        compiler_params=pltpu.CompilerParams(
            dimension_semantics=("arbitrary",)),
    )
    (idx_qk, idx_v, idx_know, loss_a, loss_k,
     gate_q, gate_k, gate_v, gate_know) = fn(
        x2d, pos_qk, pos_v, pos_know, qk_neurons, v_neurons, know_neurons,
        npt_qk, npt_v, npt_kn, W_tau_attn, bta, W_tau_know, btk)

    denom = jnp.float32(SEQ * K_CAND)
    pos_loss_attn = loss_a[0, 0] / denom
    pos_loss_know = loss_k[0, 0] / denom

    to3 = lambda a: a.reshape(1, SEQ, K_CAND)
    return (to3(gate_q), to3(gate_k), to3(gate_v), to3(idx_qk), to3(idx_v),
            pos_loss_attn, to3(gate_know), to3(idx_know), pos_loss_know)


# speedup vs baseline: 1.2227x; 1.0412x over previous
import jax
import jax.numpy as jnp
from jax.experimental import pallas as pl
from jax.experimental.pallas import tpu as pltpu

D_MODEL = 768
POS_DIM = 2
K_CAND = 24
K_TOP = 8
KEEP = 0.9
SEQ = 2048
T_BLK = 128


def _top8_threshold(eg):
    v = eg
    cum = jnp.zeros(eg.shape[:1] + (1,), jnp.float32)
    thr = jnp.zeros(eg.shape[:1] + (1,), jnp.float32)
    m1 = None
    for j in range(K_TOP):
        m = jnp.max(v, axis=1, keepdims=True)
        if j == 0:
            m1 = m
        eqm = v == m
        c = jnp.sum(jnp.where(eqm, 1.0, 0.0), axis=1, keepdims=True)
        thr = jnp.where((cum < K_TOP) & (cum + c >= K_TOP), m, thr)
        cum = cum + c
        if j < K_TOP - 1:
            v = jnp.where(eqm, -jnp.inf, v)
    return thr, m1


def _threshold_gate(scores, tau):
    raw = scores - tau
    gate = jnp.where(raw > 0, raw, 1e-08 * jnp.exp(raw))
    eg = jnp.exp(gate) - 1.0
    thr, m1 = _top8_threshold(eg)
    eg = jnp.where(eg >= thr, eg, 0.0)
    gsum = jnp.sum(eg, axis=1, keepdims=True) + 1e-08
    gstr = jnp.tanh(m1)
    return eg / gsum * gstr


def _extract(pos, npt, scores, N, unroll):
    d0 = pos[:, 0:1] - npt[0:1, :]
    d1 = pos[:, 1:2] - npt[1:2, :]
    dist = d0 * d0 + d1 * d1
    iota_nf = jax.lax.broadcasted_iota(
        jnp.int32, (T_BLK, N), 1).astype(jnp.float32)
    iota_k = jax.lax.broadcasted_iota(jnp.int32, (T_BLK, K_CAND), 1)
    bigf = jnp.float32(N)

    def step(j, carry):
        d, acc_i, acc_d, acc_s = carry
        for u in range(unroll):
            m = jnp.min(d, axis=1, keepdims=True)
            keyf = jnp.where(d == m, iota_nf, bigf)
            idxf = jnp.min(keyf, axis=1, keepdims=True)
            sel = keyf == idxf
            s = jnp.sum(jnp.where(sel, scores, 0.0), axis=1, keepdims=True)
            d = jnp.where(sel, jnp.inf, d)
            lane = iota_k == j * unroll + u
            acc_i = jnp.where(lane, idxf, acc_i)
            acc_d = jnp.where(lane, m, acc_d)
            acc_s = jnp.where(lane, s, acc_s)
        return d, acc_i, acc_d, acc_s

    init = (dist,
            jnp.zeros((T_BLK, K_CAND), jnp.float32),
            jnp.zeros((T_BLK, K_CAND), jnp.float32),
            jnp.zeros((T_BLK, K_CAND), jnp.float32))
    _, cand_if, cand_d, cand_s = jax.lax.fori_loop(
        0, K_CAND // unroll, step, init)
    return cand_if, cand_d, cand_s


def _body(x_ref, pqk_ref, pv_ref, pkn_ref, nqk_ref, nv_ref, nkn_ref,
          npqk_ref, npv_ref, npkn_ref, wta_ref, bta_ref, wtk_ref, btk_ref,
          iqk_ref, iv_ref, ikn_ref, la_ref, lk_ref,
          gq_ref, gk_ref, gv_ref, gkn_ref):
    i = pl.program_id(0)
    x = x_ref[...]
    xs = x / KEEP

    def mm(a, b_ref):
        return jax.lax.dot_general(
            a, b_ref[...], (((1,), (1,)), ((), ())),
            preferred_element_type=jnp.float32)

    tau_a = jax.lax.dot_general(
        x, wta_ref[...], (((1,), (0,)), ((), ())),
        preferred_element_type=jnp.float32) + bta_ref[...]
    tau_k = jax.lax.dot_general(
        x, wtk_ref[...], (((1,), (0,)), ((), ())),
        preferred_element_type=jnp.float32) + btk_ref[...]

    s_qk = mm(xs, nqk_ref)
    ci, cd, cs = _extract(pqk_ref[...], npqk_ref[...], s_qk, 2048, 12)
    iqk_ref[...] = ci.astype(jnp.int32)
    g_q = _threshold_gate(cs, tau_a[:, 0:1])
    gq_ref[...] = g_q
    gk_ref[...] = _threshold_gate(cs, tau_a[:, 1:2])
    part_a = jnp.sum(g_q * cd, axis=(0, 1), keepdims=True)

    s_v = mm(xs, nv_ref)
    ci, cd, cs = _extract(pv_ref[...], npv_ref[...], s_v, 2048, 12)
    iv_ref[...] = ci.astype(jnp.int32)
    g_v = _threshold_gate(cs, tau_a[:, 2:3])
    gv_ref[...] = g_v
    part_a += jnp.sum(g_v * cd, axis=(0, 1), keepdims=True)

    s_kn = mm(xs, nkn_ref)
    ci, cd, cs = _extract(pkn_ref[...], npkn_ref[...], s_kn, 4096, 12)
    ikn_ref[...] = ci.astype(jnp.int32)
    g_kn = _threshold_gate(cs, tau_k)
    gkn_ref[...] = g_kn
    part_k = jnp.sum(g_kn * cd, axis=(0, 1), keepdims=True)

    @pl.when(i == 0)
    def _():
        la_ref[...] = jnp.zeros((1, 1), jnp.float32)
        lk_ref[...] = jnp.zeros((1, 1), jnp.float32)

    la_ref[...] += part_a
    lk_ref[...] += part_k


def kernel(x, qk_neurons, v_neurons, know_neurons, neuron_pos, W_pos_qk,
           b_pos_qk, W_pos_v, b_pos_v, W_pos_know, b_pos_know, W_tau_attn,
           b_tau_attn, W_tau_know, b_tau_know, deterministic):
    del deterministic
    n_qk = qk_neurons.shape[0]
    n_v = v_neurons.shape[0]
    x2d = x.reshape(SEQ, D_MODEL)

    pos_qk = (x @ W_pos_qk + b_pos_qk).reshape(SEQ, POS_DIM)
    pos_v = (x @ W_pos_v + b_pos_v).reshape(SEQ, POS_DIM)
    pos_know = (x @ W_pos_know + b_pos_know).reshape(SEQ, POS_DIM)

    npt_qk = neuron_pos[:n_qk].T
    npt_v = neuron_pos[n_qk:n_qk + n_v].T
    npt_kn = neuron_pos[n_qk + n_v:].T
    bta = b_tau_attn.reshape(1, 3)
    btk = b_tau_know.reshape(1, 1)

    grid = (SEQ // T_BLK,)
    tok = lambda i: (i, 0)
    rep = lambda i: (0, 0)
    sd = jax.ShapeDtypeStruct
    out_shape = [
        sd((SEQ, K_CAND), jnp.int32), sd((SEQ, K_CAND), jnp.int32),
        sd((SEQ, K_CAND), jnp.int32),
        sd((1, 1), jnp.float32), sd((1, 1), jnp.float32),
        sd((SEQ, K_CAND), jnp.float32), sd((SEQ, K_CAND), jnp.float32),
        sd((SEQ, K_CAND), jnp.float32), sd((SEQ, K_CAND), jnp.float32),
    ]
    in_specs = [
        pl.BlockSpec((T_BLK, D_MODEL), tok),
        pl.BlockSpec((T_BLK, POS_DIM), tok),
        pl.BlockSpec((T_BLK, POS_DIM), tok),
        pl.BlockSpec((T_BLK, POS_DIM), tok),
        pl.BlockSpec((2048, D_MODEL), rep),
        pl.BlockSpec((2048, D_MODEL), rep),
        pl.BlockSpec((4096, D_MODEL), rep),
        pl.BlockSpec((POS_DIM, 2048), rep),
        pl.BlockSpec((POS_DIM, 2048), rep),
        pl.BlockSpec((POS_DIM, 4096), rep),
        pl.BlockSpec((D_MODEL, 3), rep),
        pl.BlockSpec((1, 3), rep),
        pl.BlockSpec((D_MODEL, 1), rep),
        pl.BlockSpec((1, 1), rep),
    ]
    out_specs = [
        pl.BlockSpec((T_BLK, K_CAND), tok), pl.BlockSpec((T_BLK, K_CAND), tok),
        pl.BlockSpec((T_BLK, K_CAND), tok),
        pl.BlockSpec((1, 1), rep), pl.BlockSpec((1, 1), rep),
        pl.BlockSpec((T_BLK, K_CAND), tok), pl.BlockSpec((T_BLK, K_CAND), tok),
        pl.BlockSpec((T_BLK, K_CAND), tok), pl.BlockSpec((T_BLK, K_CAND), tok),
    ]
    fn = pl.pallas_call(
        _body,
        grid=grid,
        in_specs=in_specs,
        out_specs=out_specs,
        out_shape=out_shape,
        compiler_params=pltpu.CompilerParams(
            dimension_semantics=("arbitrary",)),
    )
    (idx_qk, idx_v, idx_know, loss_a, loss_k,
     gate_q, gate_k, gate_v, gate_know) = fn(
        x2d, pos_qk, pos_v, pos_know, qk_neurons, v_neurons, know_neurons,
        npt_qk, npt_v, npt_kn, W_tau_attn, bta, W_tau_know, btk)

    denom = jnp.float32(SEQ * K_CAND)
    pos_loss_attn = loss_a[0, 0] / denom
    pos_loss_know = loss_k[0, 0] / denom

    to3 = lambda a: a.reshape(1, SEQ, K_CAND)
    return (to3(gate_q), to3(gate_k), to3(gate_v), to3(idx_qk), to3(idx_v),
            pos_loss_attn, to3(gate_know), to3(idx_know), pos_loss_know)
